# Initial kernel scaffold; baseline (speedup 1.0000x reference)
import jax
import jax.numpy as jnp
from jax.experimental import pallas as pl


def kernel(x, edge_index, batch, W1l, b1l, W1r, W2l, b2l, W2r, W3l, b3l, W3r):
    raise NotImplementedError("write your pallas kernel here")


# speedup vs baseline: 3.6248x; 3.6248x over previous
import functools

import jax
import jax.numpy as jnp
from jax import lax
from jax.experimental import pallas as pl
from jax.experimental.pallas import tpu as pltpu
from jax.experimental.pallas import tpu_sc as plsc

_N = 10000
_E = 320000
_G = 64
_NC = 2
_NS = 16
_L = 16
_NPAD = 10240
_RPT = _NPAD // _NS
_K = 80
_F = _G // _NC

_mesh = plsc.VectorSubcoreMesh(core_axis_name="c", subcore_axis_name="s")
_sc_params = pltpu.CompilerParams(use_tc_tiling_on_sc=False)


@functools.partial(
    pl.kernel,
    out_type=jax.ShapeDtypeStruct((_NC, _NPAD), jnp.float32),
    mesh=_mesh,
    scratch_types=[
        pltpu.VMEM((_K,), jnp.int32),
        pltpu.VMEM((_K,), jnp.float32),
        pltpu.VMEM_SHARED((_NPAD,), jnp.float32),
    ],
    compiler_params=_sc_params,
)
def _deg_kernel(dst_hbm, zeros1_hbm, out_hbm, idx_v, ones_v, deg_sh):
    c = lax.axis_index("c")
    s = lax.axis_index("s")
    for i in range(_K // _L):
        ones_v[pl.ds(i * _L, _L)] = jnp.ones((_L,), jnp.float32)
    pltpu.sync_copy(zeros1_hbm.at[pl.ds(s * _RPT, _RPT)],
                    deg_sh.at[pl.ds(s * _RPT, _RPT)])
    plsc.subcore_barrier()
    epw = _E // (_NC * _NS)
    base = (c * _NS + s) * epw

    def body(j, _):
        pltpu.sync_copy(dst_hbm.at[pl.ds(base + j * _K, _K)], idx_v)
        pltpu.sync_copy(ones_v, deg_sh.at[idx_v], add=True)
        return ()

    lax.fori_loop(0, epw // _K, body, (), unroll=False)
    plsc.subcore_barrier()
    pltpu.sync_copy(deg_sh.at[pl.ds(s * _RPT, _RPT)],
                    out_hbm.at[c, pl.ds(s * _RPT, _RPT)])


@functools.partial(
    pl.kernel,
    out_type=jax.ShapeDtypeStruct((_NC, _NPAD, _F), jnp.float32),
    mesh=_mesh,
    scratch_types=[
        pltpu.VMEM((_K,), jnp.int32),
        pltpu.VMEM((_K,), jnp.int32),
        pltpu.VMEM((_K, _F), jnp.float32),
        pltpu.VMEM_SHARED((_NPAD, _F), jnp.float32),
        pltpu.SemaphoreType.DMA,
    ],
    compiler_params=_sc_params,
)
def _pass_kernel(ta_hbm, tb_hbm, src_hbm, dst_hbm, zeros2_hbm, out_hbm,
                 di_v, si_v, rows_v, acc_sh, sem):
    c = lax.axis_index("c")
    s = lax.axis_index("s")
    pltpu.sync_copy(zeros2_hbm.at[pl.ds(s * _RPT, _RPT)],
                    acc_sh.at[pl.ds(s * _RPT, _RPT)])
    plsc.subcore_barrier()
    epw = _E // _NS
    base = s * epw

    def body(j, _):
        off = base + j * _K
        pltpu.sync_copy(dst_hbm.at[pl.ds(off, _K)], di_v)
        pltpu.sync_copy(src_hbm.at[pl.ds(off, _K)], si_v)

        @pl.when(c == 0)
        def _():
            pltpu.async_copy(ta_hbm.at[di_v], rows_v, sem).wait()

        @pl.when(c == 1)
        def _():
            pltpu.async_copy(tb_hbm.at[di_v], rows_v, sem).wait()

        pltpu.sync_copy(rows_v, acc_sh.at[si_v], add=True)
        return ()

    lax.fori_loop(0, epw // _K, body, (), unroll=False)
    plsc.subcore_barrier()
    pltpu.sync_copy(acc_sh.at[pl.ds(s * _RPT, _RPT)],
                    out_hbm.at[c, pl.ds(s * _RPT, _RPT)])


def _prep_body(batch_ref, degp_ref, u0_ref, dinv_ref, ta_ref, tb_ref):
    b = batch_ref[...]
    gids = lax.broadcasted_iota(jnp.int32, (1, _G), 1)
    onehot = jnp.where(b == gids, 1.0, 0.0)
    counts = jnp.sum(onehot, axis=0, keepdims=True)
    u0 = onehot / jnp.maximum(counts, 1.0)
    deg = degp_ref[0] + degp_ref[1]
    dinv = 1.0 / jnp.maximum(deg, 1.0)
    u0_ref[...] = u0
    dinv_ref[...] = dinv
    t = u0 * dinv
    ta_ref[...] = t[:, :_F]
    tb_ref[...] = t[:, _F:]


def _comb_body(y_ref, dinv_ref, u_ref, ta_ref, tb_ref):
    ya = y_ref[0]
    yb = y_ref[1]
    dinv = dinv_ref[...]
    u_ref[:, :_F] = ya
    u_ref[:, _F:] = yb
    ta_ref[...] = ya * dinv
    tb_ref[...] = yb * dinv


def _final_body(x_ref, u0_ref, u1_ref, u2_ref, y3_ref,
                w1l_ref, w1r_ref, w2l_ref, w2r_ref, w3l_ref, w3r_ref,
                b1l_ref, b2l_ref, b3l_ref, out_ref):
    f32 = jnp.float32

    def dgT(a, b):
        return lax.dot_general(a, b, (((1,), (1,)), ((), ())),
                               preferred_element_type=f32)

    def dgN(a, b):
        return lax.dot_general(a, b, (((0,), (0,)), ((), ())),
                               preferred_element_type=f32)

    x = x_ref[...]
    u3 = jnp.concatenate([y3_ref[0], y3_ref[1]], axis=1)
    us = (u0_ref[...], u1_ref[...], u2_ref[...], u3)
    ones = jnp.ones((_NPAD, 1), f32)
    z = [dgN(u, x) for u in us]
    sv = [dgN(u, ones) for u in us[:3]]

    w1l = w1l_ref[...]; w1r = w1r_ref[...]
    w2l = w2l_ref[...]; w2r = w2r_ref[...]
    w3l = w3l_ref[...]; w3r = w3r_ref[...]
    b1l = b1l_ref[...]; b2l = b2l_ref[...]; b3l = b3l_ref[...]

    def mm(a, b):
        return lax.dot_general(a, b, (((1,), (0,)), ((), ())),
                               preferred_element_type=f32)

    a32 = mm(w3l, w2l); b32 = mm(w3l, w2r)
    c32 = mm(w3r, w2l); d32 = mm(w3r, w2r)
    p3 = mm(a32, w1l)
    p2 = mm(a32, w1r) + mm(b32, w1l) + mm(c32, w1l)
    p1 = mm(b32, w1r) + mm(c32, w1r) + mm(d32, w1l)
    p0 = mm(d32, w1r)

    t2l = dgT(b1l, w2l)
    t2r = dgT(b1l, w2r)
    c2 = dgT(t2l, w3l)
    c1 = dgT(t2l, w3r) + dgT(t2r, w3l) + dgT(b2l, w3l)
    c0 = dgT(t2r, w3r) + dgT(b2l, w3r) + b3l

    pooled = (dgT(z[3], p3) + dgT(z[2], p2) + dgT(z[1], p1) + dgT(z[0], p0)
              + sv[2] * c2 + sv[1] * c1 + sv[0] * c0)

    m = jnp.max(pooled, axis=1, keepdims=True)
    e = pooled - m
    lse = jnp.log(jnp.sum(jnp.exp(e), axis=1, keepdims=True))
    out_ref[...] = e - lse


def _tc_call(body, out_shapes):
    return pl.pallas_call(body, out_shape=out_shapes)


def kernel(x, edge_index, batch, W1l, b1l, W1r, W2l, b2l, W2r, W3l, b3l, W3r):
    f32 = jnp.float32
    src = edge_index[0]
    dst = edge_index[1]
    zeros1 = jnp.zeros((_NPAD,), f32)
    zeros2 = jnp.zeros((_NPAD, _F), f32)

    degp = _deg_kernel(dst, zeros1)

    batch_col = jnp.concatenate(
        [batch, jnp.full((_NPAD - _N,), _G, jnp.int32)]).reshape(_NPAD, 1)
    degp_col = degp.reshape(_NC, _NPAD, 1)

    u0, dinv, ta, tb = _tc_call(
        _prep_body,
        (jax.ShapeDtypeStruct((_NPAD, _G), f32),
         jax.ShapeDtypeStruct((_NPAD, 1), f32),
         jax.ShapeDtypeStruct((_NPAD, _F), f32),
         jax.ShapeDtypeStruct((_NPAD, _F), f32)))(batch_col, degp_col)

    comb = _tc_call(
        _comb_body,
        (jax.ShapeDtypeStruct((_NPAD, _G), f32),
         jax.ShapeDtypeStruct((_NPAD, _F), f32),
         jax.ShapeDtypeStruct((_NPAD, _F), f32)))

    y1 = _pass_kernel(ta, tb, src, dst, zeros2)
    u1, ta, tb = comb(y1, dinv)
    y2 = _pass_kernel(ta, tb, src, dst, zeros2)
    u2, ta, tb = comb(y2, dinv)
    y3 = _pass_kernel(ta, tb, src, dst, zeros2)

    x_pad = jnp.concatenate([x, jnp.zeros((_NPAD - _N, x.shape[1]), f32)])
    out = _tc_call(
        _final_body,
        jax.ShapeDtypeStruct((_G, W3l.shape[0]), f32))(
            x_pad, u0, u1, u2, y3,
            W1l, W1r, W2l, W2r, W3l, W3r,
            b1l.reshape(1, -1), b2l.reshape(1, -1), b3l.reshape(1, -1))
    return out
